# baseline (speedup 1.0000x reference)
import jax
import jax.numpy as jnp
from jax.experimental import pallas as pl
from jax.experimental.pallas import tpu as pltpu

E = 8
H = 768
F = 2048
TT = 256
LB_W = 0.01


def _gate_body(x_ref, gw_ref, comb_ref, aux_ref, xb_ref):
    x = x_ref[...]
    gw = gw_ref[...]
    xb_ref[...] = x.astype(jnp.bfloat16)
    logits = jax.lax.dot_general(
        x, gw, (((1,), (1,)), ((), ())), preferred_element_type=jnp.float32)
    m = jnp.max(logits, axis=-1, keepdims=True)
    ex = jnp.exp(logits - m)
    probs = ex / jnp.sum(ex, axis=-1, keepdims=True)

    iota = jax.lax.broadcasted_iota(jnp.int32, probs.shape, 1)
    p1 = jnp.max(probs, axis=-1, keepdims=True)
    idx1 = jnp.min(jnp.where(probs == p1, iota, E), axis=-1, keepdims=True)
    oh1 = (iota == idx1)
    masked = jnp.where(oh1, -jnp.inf, probs)
    p2 = jnp.max(masked, axis=-1, keepdims=True)
    idx2 = jnp.min(jnp.where(masked == p2, iota, E), axis=-1, keepdims=True)
    oh2 = (iota == idx2)

    denom = p1 + p2 + 1e-9
    oh1f = oh1.astype(jnp.float32)
    oh2f = oh2.astype(jnp.float32)
    comb_ref[...] = (p1 / denom) * oh1f + (p2 / denom) * oh2f

    s = jnp.float32(probs.shape[0])
    f = jnp.sum(oh1f + oh2f, axis=0) / s
    pmean = jnp.sum(probs, axis=0) / s
    aux_ref[...] = jnp.reshape(LB_W * E * jnp.sum(f * pmean), (1, 1))


def _ffn_body(comb_ref, xb_ref, wgu_ref, wd_ref, out_ref, wgub_s, wdb_s):
    e = pl.program_id(0)
    t = pl.program_id(1)

    @pl.when((e == 0) & (t == 0))
    def _():
        out_ref[...] = jnp.zeros_like(out_ref)

    @pl.when(t == 0)
    def _():
        wgub_s[...] = wgu_ref[0].astype(jnp.bfloat16)
        wdb_s[...] = wd_ref[0].astype(jnp.bfloat16)

    x = xb_ref[pl.ds(t * TT, TT), :]
    gu = jnp.dot(x, wgub_s[...], preferred_element_type=jnp.float32)
    g = gu[:, :F]
    u = gu[:, F:]
    act = (g * jax.nn.sigmoid(g) * u).astype(jnp.bfloat16)
    y = jnp.dot(act, wdb_s[...], preferred_element_type=jnp.float32)
    cvals = comb_ref[pl.ds(t * TT, TT), :]
    lane = jax.lax.broadcasted_iota(jnp.int32, cvals.shape, 1)
    scale = jnp.sum(jnp.where(lane == e, cvals, 0.0), axis=1, keepdims=True)
    out_ref[pl.ds(t * TT, TT), :] += scale * y


def kernel(x, gate_w, w_gate_up, w_down):
    b, s, h = x.shape
    x_flat = x.reshape(s, h)
    nt = s // TT

    comb, aux, xb = pl.pallas_call(
        _gate_body,
        out_shape=[
            jax.ShapeDtypeStruct((s, E), jnp.float32),
            jax.ShapeDtypeStruct((1, 1), jnp.float32),
            jax.ShapeDtypeStruct((s, h), jnp.bfloat16),
        ],
    )(x_flat, gate_w)

    out = pl.pallas_call(
        _ffn_body,
        grid=(E, nt),
        in_specs=[
            pl.BlockSpec((s, E), lambda e, t: (0, 0)),
            pl.BlockSpec((s, h), lambda e, t: (0, 0)),
            pl.BlockSpec((1, H, 2 * F), lambda e, t: (e, 0, 0)),
            pl.BlockSpec((1, F, H), lambda e, t: (e, 0, 0)),
        ],
        out_specs=pl.BlockSpec((s, h), lambda e, t: (0, 0)),
        out_shape=jax.ShapeDtypeStruct((s, h), jnp.float32),
        scratch_shapes=[
            pltpu.VMEM((H, 2 * F), jnp.bfloat16),
            pltpu.VMEM((F, H), jnp.bfloat16),
        ],
        compiler_params=pltpu.CompilerParams(
            vmem_limit_bytes=120 * 1024 * 1024,
        ),
    )(comb, xb, w_gate_up, w_down)

    return out.reshape(b, s, h), aux[0, 0]


# speedup vs baseline: 1.2990x; 1.2990x over previous
import jax
import jax.numpy as jnp
from jax.experimental import pallas as pl
from jax.experimental.pallas import tpu as pltpu

E = 8
H = 768
F = 2048
TT = 1024
LB_W = 0.01


def _gate_body(x_ref, gw_ref, comb_ref, aux_ref, xb_ref):
    x = x_ref[...]
    gw = gw_ref[...]
    xb_ref[...] = x.astype(jnp.bfloat16)
    logits = jax.lax.dot_general(
        x, gw, (((1,), (1,)), ((), ())), preferred_element_type=jnp.float32)
    m = jnp.max(logits, axis=-1, keepdims=True)
    ex = jnp.exp(logits - m)
    probs = ex / jnp.sum(ex, axis=-1, keepdims=True)

    iota = jax.lax.broadcasted_iota(jnp.int32, probs.shape, 1)
    p1 = jnp.max(probs, axis=-1, keepdims=True)
    idx1 = jnp.min(jnp.where(probs == p1, iota, E), axis=-1, keepdims=True)
    oh1 = (iota == idx1)
    masked = jnp.where(oh1, -jnp.inf, probs)
    p2 = jnp.max(masked, axis=-1, keepdims=True)
    idx2 = jnp.min(jnp.where(masked == p2, iota, E), axis=-1, keepdims=True)
    oh2 = (iota == idx2)

    denom = p1 + p2 + 1e-9
    oh1f = oh1.astype(jnp.float32)
    oh2f = oh2.astype(jnp.float32)
    comb_ref[...] = (p1 / denom) * oh1f + (p2 / denom) * oh2f

    s = jnp.float32(probs.shape[0])
    f = jnp.sum(oh1f + oh2f, axis=0) / s
    pmean = jnp.sum(probs, axis=0) / s
    aux_ref[...] = jnp.reshape(LB_W * E * jnp.sum(f * pmean), (1, 1))


def _ffn_body(comb_ref, xb_ref, wgu_ref, wd_ref, out_ref):
    e = pl.program_id(0)
    t = pl.program_id(1)

    @pl.when((e == 0) & (t == 0))
    def _():
        out_ref[...] = jnp.zeros_like(out_ref)

    x = xb_ref[pl.ds(t * TT, TT), :]
    wgu = wgu_ref[0].astype(jnp.bfloat16)
    gu = jnp.dot(x, wgu, preferred_element_type=jnp.float32)
    g = gu[:, :F]
    u = gu[:, F:]
    act = (g * jax.nn.sigmoid(g) * u).astype(jnp.bfloat16)
    wd = wd_ref[0].astype(jnp.bfloat16)
    y = jnp.dot(act, wd, preferred_element_type=jnp.float32)
    cvals = comb_ref[pl.ds(t * TT, TT), :]
    lane = jax.lax.broadcasted_iota(jnp.int32, cvals.shape, 1)
    scale = jnp.sum(jnp.where(lane == e, cvals, 0.0), axis=1, keepdims=True)
    out_ref[pl.ds(t * TT, TT), :] += scale * y


def kernel(x, gate_w, w_gate_up, w_down):
    b, s, h = x.shape
    x_flat = x.reshape(s, h)
    nt = s // TT

    comb, aux, xb = pl.pallas_call(
        _gate_body,
        out_shape=[
            jax.ShapeDtypeStruct((s, E), jnp.float32),
            jax.ShapeDtypeStruct((1, 1), jnp.float32),
            jax.ShapeDtypeStruct((s, h), jnp.bfloat16),
        ],
    )(x_flat, gate_w)

    out = pl.pallas_call(
        _ffn_body,
        grid=(E, nt),
        in_specs=[
            pl.BlockSpec((s, E), lambda e, t: (0, 0)),
            pl.BlockSpec((s, h), lambda e, t: (0, 0)),
            pl.BlockSpec((1, H, 2 * F), lambda e, t: (e, 0, 0)),
            pl.BlockSpec((1, F, H), lambda e, t: (e, 0, 0)),
        ],
        out_specs=pl.BlockSpec((s, h), lambda e, t: (0, 0)),
        out_shape=jax.ShapeDtypeStruct((s, h), jnp.float32),
        compiler_params=pltpu.CompilerParams(
            vmem_limit_bytes=120 * 1024 * 1024,
        ),
    )(comb, xb, w_gate_up, w_down)

    return out.reshape(b, s, h), aux[0, 0]


# speedup vs baseline: 1.4431x; 1.1110x over previous
import functools

import jax
import jax.numpy as jnp
from jax import lax
from jax.experimental import pallas as pl
from jax.experimental.pallas import tpu as pltpu
from jax.experimental.pallas import tpu_sc as plsc

E = 8
H = 768
F = 2048
TT = 512
NTP = 16
PR = NTP * TT
LB_W = 0.01


def _gate_body(x_ref, gw_ref, pos1_ref, pos2_ref, w1x_ref, w2x_ref,
               te_ref, send_ref, aux_ref):
    x = x_ref[...]
    gw = gw_ref[...]
    s = x.shape[0]
    logits = jax.lax.dot_general(
        x, gw, (((1,), (1,)), ((), ())), preferred_element_type=jnp.float32)
    m = jnp.max(logits, axis=-1, keepdims=True)
    ex = jnp.exp(logits - m)
    probs = ex / jnp.sum(ex, axis=-1, keepdims=True)

    iota = jax.lax.broadcasted_iota(jnp.int32, probs.shape, 1)
    p1 = jnp.max(probs, axis=-1, keepdims=True)
    idx1 = jnp.min(jnp.where(probs == p1, iota, E), axis=-1, keepdims=True)
    oh1 = (iota == idx1)
    masked = jnp.where(oh1, -jnp.inf, probs)
    p2 = jnp.max(masked, axis=-1, keepdims=True)
    idx2 = jnp.min(jnp.where(masked == p2, iota, E), axis=-1, keepdims=True)
    oh2 = (iota == idx2)

    denom = p1 + p2 + 1e-9
    oh1f = oh1.astype(jnp.float32)
    oh2f = oh2.astype(jnp.float32)
    w1x_ref[...] = jnp.broadcast_to(p1 / denom, (s, 128))
    w2x_ref[...] = jnp.broadcast_to(p2 / denom, (s, 128))

    sf = jnp.float32(s)
    f = jnp.sum(oh1f + oh2f, axis=0) / sf
    pmean = jnp.sum(probs, axis=0) / sf
    aux_ref[...] = jnp.reshape(LB_W * E * jnp.sum(f * pmean), (1, 1))

    O = jnp.concatenate([oh1f, oh2f], axis=0)
    a2 = 2 * s
    ri = jax.lax.broadcasted_iota(jnp.int32, (a2, a2), 0)
    ci = jax.lax.broadcasted_iota(jnp.int32, (a2, a2), 1)
    L = (ci < ri).astype(jnp.bfloat16)
    rank = jax.lax.dot_general(
        L, O.astype(jnp.bfloat16), (((1,), (0,)), ((), ())),
        preferred_element_type=jnp.float32)

    counts = jnp.sum(O, axis=0, keepdims=True)
    pcounts = jnp.ceil(counts / TT) * TT
    ce = jax.lax.broadcasted_iota(jnp.int32, (E, E), 0)
    cc = jax.lax.broadcasted_iota(jnp.int32, (E, E), 1)
    tri = (ce < cc).astype(jnp.float32)
    offs_pad = jax.lax.dot_general(
        pcounts, tri, (((1,), (0,)), ((), ())),
        preferred_element_type=jnp.float32)

    pos = jnp.sum(O * (rank + offs_pad), axis=1, keepdims=True)
    posi = pos.astype(jnp.int32)
    pos1_ref[...] = posi[:s]
    pos2_ref[...] = posi[s:]

    tstart = (jax.lax.broadcasted_iota(jnp.int32, (1, NTP), 1) * TT
              ).astype(jnp.float32)
    op_col = jnp.broadcast_to(offs_pad.reshape(E, 1), (E, NTP))
    te = jnp.sum((op_col <= tstart).astype(jnp.float32), axis=0,
                 keepdims=True) - 1.0
    seg_end = offs_pad + counts
    te_b = jnp.broadcast_to(te, (E, NTP))
    e_col = jax.lax.broadcasted_iota(jnp.int32, (E, NTP), 0).astype(jnp.float32)
    send = jnp.sum(jnp.where(te_b == e_col,
                             jnp.broadcast_to(seg_end.reshape(E, 1), (E, NTP)),
                             0.0), axis=0, keepdims=True)
    te_ref[...] = te.astype(jnp.int32)
    send_ref[...] = send.astype(jnp.int32)


def _ffn_body(te_ref, send_ref, xg_ref, ws_ref, wgu_ref, wd_ref, y_ref,
              wgub_s, wdb_s):
    n = pl.program_id(0)
    e = te_ref[n]
    seg_end = send_ref[n]
    prev_e = jnp.where(n == 0, -1, te_ref[jnp.maximum(n - 1, 0)])

    @pl.when(e != prev_e)
    def _():
        wgub_s[...] = wgu_ref[0].astype(jnp.bfloat16)
        wdb_s[...] = wd_ref[0].astype(jnp.bfloat16)

    used = seg_end > n * TT

    @pl.when(used)
    def _():
        ri = jax.lax.broadcasted_iota(jnp.int32, (TT, 1), 0) + n * TT
        rmask = ri < seg_end
        x = jnp.where(rmask, xg_ref[...], 0.0).astype(jnp.bfloat16)
        gu = jnp.dot(x, wgub_s[...], preferred_element_type=jnp.float32)
        g = gu[:, :F]
        u = gu[:, F:]
        act = (g * jax.nn.sigmoid(g) * u).astype(jnp.bfloat16)
        y = jnp.dot(act, wdb_s[...], preferred_element_type=jnp.float32)
        ws = jnp.where(rmask, ws_ref[:, 0:1], 0.0)
        y_ref[...] = y * ws

    @pl.when(jnp.logical_not(used))
    def _():
        y_ref[...] = jnp.zeros_like(y_ref)


_info = plsc.get_sparse_core_info()
_NC = _info.num_cores
_NS = _info.num_subcores
_NW = _NC * _NS


def _make_scatter(s, h):
    tpw = s // _NW
    mesh = plsc.VectorSubcoreMesh(core_axis_name="c", subcore_axis_name="s")

    @functools.partial(
        pl.kernel, mesh=mesh,
        out_type=[
            jax.ShapeDtypeStruct((PR, h), jnp.float32),
            jax.ShapeDtypeStruct((PR, 128), jnp.float32),
        ],
        scratch_types=[
            pltpu.VMEM((tpw,), jnp.int32),
            pltpu.VMEM((tpw,), jnp.int32),
            pltpu.VMEM((tpw, h), jnp.float32),
            pltpu.VMEM((tpw, 128), jnp.float32),
            pltpu.VMEM((tpw, 128), jnp.float32),
            pltpu.SemaphoreType.DMA,
        ],
    )
    def k(x_hbm, pos1_hbm, pos2_hbm, w1x_hbm, w2x_hbm, xg_hbm, ws_hbm,
          idx1_v, idx2_v, xv, w1v, w2v, sem):
        wid = lax.axis_index("s") * _NC + lax.axis_index("c")
        base = wid * tpw
        pltpu.sync_copy(pos1_hbm.at[pl.ds(base, tpw)], idx1_v)
        pltpu.sync_copy(pos2_hbm.at[pl.ds(base, tpw)], idx2_v)
        pltpu.sync_copy(x_hbm.at[pl.ds(base, tpw), :], xv)
        pltpu.sync_copy(w1x_hbm.at[pl.ds(base, tpw), :], w1v)
        pltpu.sync_copy(w2x_hbm.at[pl.ds(base, tpw), :], w2v)
        c1 = pltpu.async_copy(xv, xg_hbm.at[idx1_v], sem)
        c2 = pltpu.async_copy(xv, xg_hbm.at[idx2_v], sem)
        c3 = pltpu.async_copy(w1v, ws_hbm.at[idx1_v], sem)
        c4 = pltpu.async_copy(w2v, ws_hbm.at[idx2_v], sem)
        c1.wait()
        c2.wait()
        c3.wait()
        c4.wait()

    return k


def _make_combine(s, h):
    tpw = s // _NW
    half = tpw // 2
    mesh = plsc.VectorSubcoreMesh(core_axis_name="c", subcore_axis_name="s")

    @functools.partial(
        pl.kernel, mesh=mesh,
        out_type=jax.ShapeDtypeStruct((s, h), jnp.float32),
        scratch_types=[
            pltpu.VMEM((half,), jnp.int32),
            pltpu.VMEM((half,), jnp.int32),
            pltpu.VMEM((half, h), jnp.float32),
            pltpu.VMEM((half, h), jnp.float32),
            pltpu.SemaphoreType.DMA,
        ],
    )
    def k(y_hbm, pos1_hbm, pos2_hbm, out_hbm, i1, i2, r1, r2, sem):
        wid = lax.axis_index("s") * _NC + lax.axis_index("c")

        def chunk(ci, carry):
            base = wid * tpw + ci * half
            pltpu.sync_copy(pos1_hbm.at[pl.ds(base, half)], i1)
            pltpu.sync_copy(pos2_hbm.at[pl.ds(base, half)], i2)
            pltpu.async_copy(y_hbm.at[i1], r1, sem).wait()
            pltpu.async_copy(y_hbm.at[i2], r2, sem).wait()

            def row(r, carry2):
                for c in range(h // 16):
                    sl = pl.ds(c * 16, 16)
                    r1[r, sl] = r1[r, sl] + r2[r, sl]
                return carry2

            lax.fori_loop(0, half, row, 0)
            pltpu.sync_copy(r1, out_hbm.at[pl.ds(base, half), :])
            return carry

        lax.fori_loop(0, 2, chunk, 0)

    return k


def kernel(x, gate_w, w_gate_up, w_down):
    b, s, h = x.shape
    x_flat = x.reshape(s, h)

    pos1, pos2, w1x, w2x, te, send, aux = pl.pallas_call(
        _gate_body,
        out_shape=[
            jax.ShapeDtypeStruct((s, 1), jnp.int32),
            jax.ShapeDtypeStruct((s, 1), jnp.int32),
            jax.ShapeDtypeStruct((s, 128), jnp.float32),
            jax.ShapeDtypeStruct((s, 128), jnp.float32),
            jax.ShapeDtypeStruct((1, NTP), jnp.int32),
            jax.ShapeDtypeStruct((1, NTP), jnp.int32),
            jax.ShapeDtypeStruct((1, 1), jnp.float32),
        ],
        compiler_params=pltpu.CompilerParams(
            vmem_limit_bytes=120 * 1024 * 1024,
        ),
    )(x_flat, gate_w)

    p1 = pos1.reshape(s)
    p2 = pos2.reshape(s)
    xg, wsort = _make_scatter(s, h)(x_flat, p1, p2, w1x, w2x)

    grid_spec = pltpu.PrefetchScalarGridSpec(
        num_scalar_prefetch=2,
        grid=(NTP,),
        in_specs=[
            pl.BlockSpec((TT, H), lambda n, te, send: (n, 0)),
            pl.BlockSpec((TT, 128), lambda n, te, send: (n, 0)),
            pl.BlockSpec((1, H, 2 * F), lambda n, te, send: (te[n], 0, 0)),
            pl.BlockSpec((1, F, H), lambda n, te, send: (te[n], 0, 0)),
        ],
        out_specs=pl.BlockSpec((TT, H), lambda n, te, send: (n, 0)),
        scratch_shapes=[
            pltpu.VMEM((H, 2 * F), jnp.bfloat16),
            pltpu.VMEM((F, H), jnp.bfloat16),
        ],
    )
    y = pl.pallas_call(
        _ffn_body,
        grid_spec=grid_spec,
        out_shape=jax.ShapeDtypeStruct((PR, H), jnp.float32),
        compiler_params=pltpu.CompilerParams(
            vmem_limit_bytes=120 * 1024 * 1024,
        ),
    )(te.reshape(NTP), send.reshape(NTP), xg, wsort, w_gate_up, w_down)

    out = _make_combine(s, h)(y, p1, p2)
    return out.reshape(b, s, h), aux[0, 0]


# speedup vs baseline: 1.4541x; 1.0076x over previous
import functools

import jax
import jax.numpy as jnp
from jax import lax
from jax.experimental import pallas as pl
from jax.experimental.pallas import tpu as pltpu
from jax.experimental.pallas import tpu_sc as plsc

E = 8
H = 768
F = 2048
TT = 768
NTP = 14
PR = NTP * TT
LB_W = 0.01


def _gate_body(x_ref, gw_ref, pos1_ref, pos2_ref, w1x_ref, w2x_ref,
               te_ref, send_ref, aux_ref):
    x = x_ref[...]
    gw = gw_ref[...]
    s = x.shape[0]
    logits = jax.lax.dot_general(
        x, gw, (((1,), (1,)), ((), ())), preferred_element_type=jnp.float32)
    m = jnp.max(logits, axis=-1, keepdims=True)
    ex = jnp.exp(logits - m)
    probs = ex / jnp.sum(ex, axis=-1, keepdims=True)

    iota = jax.lax.broadcasted_iota(jnp.int32, probs.shape, 1)
    p1 = jnp.max(probs, axis=-1, keepdims=True)
    idx1 = jnp.min(jnp.where(probs == p1, iota, E), axis=-1, keepdims=True)
    oh1 = (iota == idx1)
    masked = jnp.where(oh1, -jnp.inf, probs)
    p2 = jnp.max(masked, axis=-1, keepdims=True)
    idx2 = jnp.min(jnp.where(masked == p2, iota, E), axis=-1, keepdims=True)
    oh2 = (iota == idx2)

    denom = p1 + p2 + 1e-9
    oh1f = oh1.astype(jnp.float32)
    oh2f = oh2.astype(jnp.float32)
    w1x_ref[...] = jnp.broadcast_to(p1 / denom, (s, 128))
    w2x_ref[...] = jnp.broadcast_to(p2 / denom, (s, 128))

    sf = jnp.float32(s)
    f = jnp.sum(oh1f + oh2f, axis=0) / sf
    pmean = jnp.sum(probs, axis=0) / sf
    aux_ref[...] = jnp.reshape(LB_W * E * jnp.sum(f * pmean), (1, 1))

    O = jnp.concatenate([oh1f, oh2f], axis=0)
    a2 = 2 * s
    ri = jax.lax.broadcasted_iota(jnp.int32, (a2, a2), 0)
    ci = jax.lax.broadcasted_iota(jnp.int32, (a2, a2), 1)
    L = (ci < ri).astype(jnp.bfloat16)
    rank = jax.lax.dot_general(
        L, O.astype(jnp.bfloat16), (((1,), (0,)), ((), ())),
        preferred_element_type=jnp.float32)

    counts = jnp.sum(O, axis=0, keepdims=True)
    pcounts = jnp.ceil(counts / TT) * TT
    ce = jax.lax.broadcasted_iota(jnp.int32, (E, E), 0)
    cc = jax.lax.broadcasted_iota(jnp.int32, (E, E), 1)
    tri = (ce < cc).astype(jnp.float32)
    offs_pad = jax.lax.dot_general(
        pcounts, tri, (((1,), (0,)), ((), ())),
        preferred_element_type=jnp.float32)

    pos = jnp.sum(O * (rank + offs_pad), axis=1, keepdims=True)
    posi = pos.astype(jnp.int32)
    pos1_ref[...] = posi[:s]
    pos2_ref[...] = posi[s:]

    tstart = (jax.lax.broadcasted_iota(jnp.int32, (1, NTP), 1) * TT
              ).astype(jnp.float32)
    op_col = jnp.broadcast_to(offs_pad.reshape(E, 1), (E, NTP))
    te = jnp.sum((op_col <= tstart).astype(jnp.float32), axis=0,
                 keepdims=True) - 1.0
    seg_end = offs_pad + counts
    te_b = jnp.broadcast_to(te, (E, NTP))
    e_col = jax.lax.broadcasted_iota(jnp.int32, (E, NTP), 0).astype(jnp.float32)
    send = jnp.sum(jnp.where(te_b == e_col,
                             jnp.broadcast_to(seg_end.reshape(E, 1), (E, NTP)),
                             0.0), axis=0, keepdims=True)
    te_ref[...] = te.astype(jnp.int32)
    send_ref[...] = send.astype(jnp.int32)


def _ffn_body(te_ref, send_ref, xg_ref, ws_ref, wgu_ref, wd_ref, y_ref,
              wgub_s, wdb_s):
    n = pl.program_id(0)
    e = te_ref[n]
    seg_end = send_ref[n]
    prev_e = jnp.where(n == 0, -1, te_ref[jnp.maximum(n - 1, 0)])

    @pl.when(e != prev_e)
    def _():
        wgub_s[...] = wgu_ref[0].astype(jnp.bfloat16)
        wdb_s[...] = wd_ref[0].astype(jnp.bfloat16)

    used = seg_end > n * TT

    @pl.when(used)
    def _():
        ri = jax.lax.broadcasted_iota(jnp.int32, (TT, 1), 0) + n * TT
        rmask = ri < seg_end
        x = jnp.where(rmask, xg_ref[...], 0.0).astype(jnp.bfloat16)
        gu = jnp.dot(x, wgub_s[...], preferred_element_type=jnp.float32)
        g = gu[:, :F]
        u = gu[:, F:]
        act = (g * jax.nn.sigmoid(g) * u).astype(jnp.bfloat16)
        y = jnp.dot(act, wdb_s[...], preferred_element_type=jnp.float32)
        ws = jnp.where(rmask, ws_ref[:, 0:1], 0.0)
        y_ref[...] = y * ws

    @pl.when(jnp.logical_not(used))
    def _():
        y_ref[...] = jnp.zeros_like(y_ref)


_info = plsc.get_sparse_core_info()
_NC = _info.num_cores
_NS = _info.num_subcores
_NW = _NC * _NS


def _make_scatter(s, h):
    tpw = s // _NW
    mesh = plsc.VectorSubcoreMesh(core_axis_name="c", subcore_axis_name="s")

    @functools.partial(
        pl.kernel, mesh=mesh,
        out_type=[
            jax.ShapeDtypeStruct((PR, h), jnp.float32),
            jax.ShapeDtypeStruct((PR, 128), jnp.float32),
        ],
        scratch_types=[
            pltpu.VMEM((tpw,), jnp.int32),
            pltpu.VMEM((tpw,), jnp.int32),
            pltpu.VMEM((tpw, h), jnp.float32),
            pltpu.VMEM((tpw, 128), jnp.float32),
            pltpu.VMEM((tpw, 128), jnp.float32),
            pltpu.SemaphoreType.DMA,
        ],
    )
    def k(x_hbm, pos1_hbm, pos2_hbm, w1x_hbm, w2x_hbm, xg_hbm, ws_hbm,
          idx1_v, idx2_v, xv, w1v, w2v, sem):
        wid = lax.axis_index("s") * _NC + lax.axis_index("c")
        base = wid * tpw
        pltpu.sync_copy(pos1_hbm.at[pl.ds(base, tpw)], idx1_v)
        pltpu.sync_copy(pos2_hbm.at[pl.ds(base, tpw)], idx2_v)
        pltpu.sync_copy(x_hbm.at[pl.ds(base, tpw), :], xv)
        pltpu.sync_copy(w1x_hbm.at[pl.ds(base, tpw), :], w1v)
        pltpu.sync_copy(w2x_hbm.at[pl.ds(base, tpw), :], w2v)
        c1 = pltpu.async_copy(xv, xg_hbm.at[idx1_v], sem)
        c2 = pltpu.async_copy(xv, xg_hbm.at[idx2_v], sem)
        c3 = pltpu.async_copy(w1v, ws_hbm.at[idx1_v], sem)
        c4 = pltpu.async_copy(w2v, ws_hbm.at[idx2_v], sem)
        c1.wait()
        c2.wait()
        c3.wait()
        c4.wait()

    return k


def _make_combine(s, h):
    tpw = s // _NW
    half = tpw // 2
    mesh = plsc.VectorSubcoreMesh(core_axis_name="c", subcore_axis_name="s")

    @functools.partial(
        pl.kernel, mesh=mesh,
        out_type=jax.ShapeDtypeStruct((s, h), jnp.float32),
        scratch_types=[
            pltpu.VMEM((half,), jnp.int32),
            pltpu.VMEM((half,), jnp.int32),
            pltpu.VMEM((half, h), jnp.float32),
            pltpu.VMEM((half, h), jnp.float32),
            pltpu.SemaphoreType.DMA,
        ],
    )
    def k(y_hbm, pos1_hbm, pos2_hbm, out_hbm, i1, i2, r1, r2, sem):
        wid = lax.axis_index("s") * _NC + lax.axis_index("c")

        def chunk(ci, carry):
            base = wid * tpw + ci * half
            pltpu.sync_copy(pos1_hbm.at[pl.ds(base, half)], i1)
            pltpu.sync_copy(pos2_hbm.at[pl.ds(base, half)], i2)
            pltpu.async_copy(y_hbm.at[i1], r1, sem).wait()
            pltpu.async_copy(y_hbm.at[i2], r2, sem).wait()

            def row(r, carry2):
                for c in range(h // 16):
                    sl = pl.ds(c * 16, 16)
                    r1[r, sl] = r1[r, sl] + r2[r, sl]
                return carry2

            lax.fori_loop(0, half, row, 0)
            pltpu.sync_copy(r1, out_hbm.at[pl.ds(base, half), :])
            return carry

        lax.fori_loop(0, 2, chunk, 0)

    return k


def kernel(x, gate_w, w_gate_up, w_down):
    b, s, h = x.shape
    x_flat = x.reshape(s, h)

    pos1, pos2, w1x, w2x, te, send, aux = pl.pallas_call(
        _gate_body,
        out_shape=[
            jax.ShapeDtypeStruct((s, 1), jnp.int32),
            jax.ShapeDtypeStruct((s, 1), jnp.int32),
            jax.ShapeDtypeStruct((s, 128), jnp.float32),
            jax.ShapeDtypeStruct((s, 128), jnp.float32),
            jax.ShapeDtypeStruct((1, NTP), jnp.int32),
            jax.ShapeDtypeStruct((1, NTP), jnp.int32),
            jax.ShapeDtypeStruct((1, 1), jnp.float32),
        ],
        compiler_params=pltpu.CompilerParams(
            vmem_limit_bytes=120 * 1024 * 1024,
        ),
    )(x_flat, gate_w)

    p1 = pos1.reshape(s)
    p2 = pos2.reshape(s)
    xg, wsort = _make_scatter(s, h)(x_flat, p1, p2, w1x, w2x)

    grid_spec = pltpu.PrefetchScalarGridSpec(
        num_scalar_prefetch=2,
        grid=(NTP,),
        in_specs=[
            pl.BlockSpec((TT, H), lambda n, te, send: (n, 0)),
            pl.BlockSpec((TT, 128), lambda n, te, send: (n, 0)),
            pl.BlockSpec((1, H, 2 * F), lambda n, te, send: (te[n], 0, 0)),
            pl.BlockSpec((1, F, H), lambda n, te, send: (te[n], 0, 0)),
        ],
        out_specs=pl.BlockSpec((TT, H), lambda n, te, send: (n, 0)),
        scratch_shapes=[
            pltpu.VMEM((H, 2 * F), jnp.bfloat16),
            pltpu.VMEM((F, H), jnp.bfloat16),
        ],
    )
    y = pl.pallas_call(
        _ffn_body,
        grid_spec=grid_spec,
        out_shape=jax.ShapeDtypeStruct((PR, H), jnp.float32),
        compiler_params=pltpu.CompilerParams(
            vmem_limit_bytes=120 * 1024 * 1024,
        ),
    )(te.reshape(NTP), send.reshape(NTP), xg, wsort, w_gate_up, w_down)

    out = _make_combine(s, h)(y, p1, p2)
    return out.reshape(b, s, h), aux[0, 0]
